# baseline (speedup 1.0000x reference)
import jax
import jax.numpy as jnp
from jax import lax
from jax.experimental import pallas as pl
from jax.experimental.pallas import tpu as pltpu


def _qhead_kernel(x_ref, w_ref, b_ref, o_ref):
    acc = lax.dot_general(
        w_ref[...], x_ref[...],
        dimension_numbers=(((1,), (0,)), ((), ())),
        preferred_element_type=jnp.float32,
    )
    b_col = jnp.transpose(b_ref[...])
    o_ref[...] = (acc + b_col).astype(o_ref.dtype)


def kernel(x, weight, bias):
    B, K = x.shape
    N = weight.shape[0]

    xt = x.T

    tn = 16384
    while B % tn:
        tn //= 2
    grid = (B // tn,)

    outt = pl.pallas_call(
        _qhead_kernel,
        out_shape=jax.ShapeDtypeStruct((N, B), x.dtype),
        grid=grid,
        in_specs=[
            pl.BlockSpec((K, tn), lambda i: (0, i)),
            pl.BlockSpec((N, K), lambda i: (0, 0)),
            pl.BlockSpec((1, N), lambda i: (0, 0)),
        ],
        out_specs=pl.BlockSpec((N, tn), lambda i: (0, i)),
        compiler_params=pltpu.CompilerParams(
            dimension_semantics=("parallel",),
            vmem_limit_bytes=64 * 1024 * 1024,
        ),
    )(xt, weight, bias.reshape(1, N))

    return outt.T


# speedup vs baseline: 1.3144x; 1.3144x over previous
import jax
import jax.numpy as jnp
from jax import lax
from jax.experimental import pallas as pl
from jax.experimental.pallas import tpu as pltpu


def _qhead_kernel(x_ref, w_ref, b_ref, o_ref):
    acc = lax.dot_general(
        w_ref[...], x_ref[...],
        dimension_numbers=(((1,), (0,)), ((), ())),
        preferred_element_type=jnp.float32,
    )
    b_col = jnp.transpose(b_ref[...])
    o_ref[...] = (acc + b_col).astype(o_ref.dtype)


def kernel(x, weight, bias):
    B, K = x.shape
    N = weight.shape[0]

    xt = x.T

    tn = 65536
    while B % tn:
        tn //= 2
    grid = (B // tn,)

    outt = pl.pallas_call(
        _qhead_kernel,
        out_shape=jax.ShapeDtypeStruct((N, B), x.dtype),
        grid=grid,
        in_specs=[
            pl.BlockSpec((K, tn), lambda i: (0, i)),
            pl.BlockSpec((N, K), lambda i: (0, 0)),
            pl.BlockSpec((1, N), lambda i: (0, 0)),
        ],
        out_specs=pl.BlockSpec((N, tn), lambda i: (0, i)),
        compiler_params=pltpu.CompilerParams(
            dimension_semantics=("parallel",),
            vmem_limit_bytes=64 * 1024 * 1024,
        ),
    )(xt, weight, bias.reshape(1, N))

    return outt.T
